# baseline (speedup 1.0000x reference)
import functools

import jax
import jax.numpy as jnp
from jax import lax
from jax.experimental import pallas as pl
from jax.experimental.pallas import tpu as pltpu
from jax.experimental.pallas import tpu_sc as plsc

HID = 128
N = 10000
E = 320000
C = 2500
G = 64
NC = N + C

_CH = 1250

_dot = functools.partial(jnp.dot, preferred_element_type=jnp.float32,
                         precision=jax.lax.Precision.HIGHEST)
_dotd = functools.partial(jnp.dot, preferred_element_type=jnp.float32,
                          precision=jax.lax.Precision.DEFAULT)


def _dot2(oh, x):
    xh = x.astype(jnp.bfloat16).astype(jnp.float32)
    return _dotd(oh, xh) + _dotd(oh, x - xh)


def _f32(s):
    return jax.ShapeDtypeStruct(s, jnp.float32)


def _bn_small(x):
    mu = jnp.mean(x, axis=0, keepdims=True)
    var = jnp.mean(x * x, axis=0, keepdims=True) - mu * mu
    return (x - mu) / jnp.sqrt(var + 1e-5) + 1e-4


def _stats(read_chunk, nrows):
    ssum = jnp.zeros((1, HID), jnp.float32)
    ssq = jnp.zeros((1, HID), jnp.float32)
    for k in range(nrows // _CH):
        c = read_chunk(k)
        ssum = ssum + jnp.sum(c, axis=0, keepdims=True)
        ssq = ssq + jnp.sum(c * c, axis=0, keepdims=True)
    mu = ssum / nrows
    var = ssq / nrows - mu * mu
    return mu, 1.0 / jnp.sqrt(var + 1e-5)


_AB = 1000


def _head_body(x_ref, embf_ref, n2c_ref, h_ref, cq_ref, cs_ref):
    k = pl.program_id(0)
    iota120 = jax.lax.broadcasted_iota(jnp.int32, (1, 120), 1)
    h = jnp.zeros((_AB, HID), jnp.float32)
    for i in range(9):
        oh = (x_ref[:, i:i + 1] == iota120).astype(jnp.float32)
        h = h + _dot2(oh, embf_ref[i * 120:(i + 1) * 120, :])
    h_ref[...] = h
    iotaC = jax.lax.broadcasted_iota(jnp.int32, (C, 1), 0)
    ohc = (iotaC == n2c_ref[0]).astype(jnp.float32)

    @pl.when(k == 0)
    def _():
        cq_ref[...] = jnp.zeros((C, HID), jnp.float32)
        cs_ref[...] = jnp.zeros((C, 1), jnp.float32)

    cq_ref[...] += _dot2(ohc, h)
    cs_ref[...] += jnp.sum(ohc, axis=1, keepdims=True)


def _head(x, emb_flat, n2c_blocks):
    return pl.pallas_call(
        _head_body,
        grid=(N // _AB,),
        in_specs=[pl.BlockSpec((_AB, 9), lambda k: (k, 0)),
                  pl.BlockSpec((9 * 120, HID), lambda k: (0, 0)),
                  pl.BlockSpec((1, 1, _AB), lambda k: (k, 0, 0))],
        out_specs=[pl.BlockSpec((_AB, HID), lambda k: (k, 0)),
                   pl.BlockSpec((C, HID), lambda k: (0, 0)),
                   pl.BlockSpec((C, 1), lambda k: (0, 0))],
        out_shape=(_f32((N, HID)), _f32((C, HID)), _f32((C, 1))),
    )(x, emb_flat, n2c_blocks)


def _tc(body, out_shape, *args, scratch_shapes=()):
    return pl.pallas_call(body, out_shape=out_shape,
                          scratch_shapes=list(scratch_shapes))(*args)


def _norm_matmul_out(src_ref, mu, inv, dinv_ref, W_ref, g_ref, nrows):
    for k in range(nrows // _CH):
        sl = slice(k * _CH, (k + 1) * _CH)
        xb = (src_ref[sl, :] - mu) * inv + 1e-4
        h1 = _dot(xb, W_ref[...])
        g_ref[sl, :] = dinv_ref[sl, :] * h1


def _pre_body(xa_ref, dinv_ref, W_ref, g_ref):
    mu, inv = _stats(lambda k: xa_ref[k * _CH:(k + 1) * _CH, :], NC)
    _norm_matmul_out(xa_ref, mu, inv, dinv_ref, W_ref, g_ref, NC)


def _mid_body(xa_ref, acc_ref, dinv_ref, bprev_ref, W_ref, xa2_ref, g_ref):
    b = bprev_ref[...]
    for k in range(NC // _CH):
        sl = slice(k * _CH, (k + 1) * _CH)
        xa2_ref[sl, :] = xa_ref[sl, :] + jax.nn.relu(
            dinv_ref[sl, :] * acc_ref[sl, :] + b)
    mu, inv = _stats(lambda k: xa2_ref[k * _CH:(k + 1) * _CH, :], NC)
    _norm_matmul_out(xa2_ref, mu, inv, dinv_ref, W_ref, g_ref, NC)


def _mask_body(xa_ref, acc_ref, dinv_ref, bprev_ref,
               aW1_ref, ab1_ref, aW2_ref, ab2_ref,
               bohn_ref,
               xn_ref, mm_ref, eps_ref):
    b = bprev_ref[...]
    mo_parts = []
    for k in range(NC // _CH):
        sl = slice(k * _CH, (k + 1) * _CH)
        out = xa_ref[sl, :] + jax.nn.relu(
            dinv_ref[sl, :] * acc_ref[sl, :] + b)
        if k < N // _CH:
            xn_ref[sl, :] = out
        else:
            mo_parts.append(out)
    mo = jnp.concatenate(mo_parts, axis=0)
    hm = _bn_small(mo)
    hm = jax.nn.relu(_dot(hm, aW1_ref[...]) + ab1_ref[...])
    hm = _bn_small(hm)
    att = _dot(hm, aW2_ref[...]) + ab2_ref[...]
    a0 = att[:, 0:1]
    a1 = att[:, 1:2]
    m = jnp.maximum(a0, a1)
    e0 = jnp.exp(a0 - m)
    e1 = jnp.exp(a1 - m)
    mm0 = e0 / (e0 + e1)
    mm_ref[...] = jnp.concatenate([mm0, 1.0 - mm0], axis=1)
    gsum = jnp.zeros((G, HID), jnp.float32)
    cnt = jnp.zeros((G, 1), jnp.float32)
    for k in range(N // _CH):
        sl = slice(k * _CH, (k + 1) * _CH)
        gsum = gsum + _dot2(bohn_ref[:, sl], xn_ref[sl, :])
        cnt = cnt + jnp.sum(bohn_ref[:, sl], axis=1, keepdims=True)
    gmean = gsum / jnp.maximum(cnt, 1.0)
    e = jnp.mean(gmean, axis=-1, keepdims=True)
    for k in range(N // _CH):
        sl = slice(k * _CH, (k + 1) * _CH)
        eps_ref[sl, :] = jax.lax.dot_general(
            bohn_ref[:, sl], e, (((0,), (0,)), ((), ())),
            preferred_element_type=jnp.float32,
            precision=jax.lax.Precision.HIGHEST)


def _prec2_body(xn_ref, sc3_ref, dinv_ref, ctxW_ref, cauW_ref,
                gs_ref, gc_ref, xs_sc, xc_sc):
    for k in range(N // _CH):
        sl = slice(k * _CH, (k + 1) * _CH)
        xn = xn_ref[sl, :]
        m0 = sc3_ref[sl, 0:1]
        m1 = sc3_ref[sl, 1:2]
        eps = sc3_ref[sl, 2:3]
        xs_sc[sl, :] = m0 * xn + (1.0 - m0) * eps
        xc_sc[sl, :] = m1 * xn + (1.0 - m1) * eps
    mu_s, inv_s = _stats(lambda k: xs_sc[k * _CH:(k + 1) * _CH, :], N)
    _norm_matmul_out(xs_sc, mu_s, inv_s, dinv_ref, ctxW_ref, gs_ref, N)
    mu_c, inv_c = _stats(lambda k: xc_sc[k * _CH:(k + 1) * _CH, :], N)
    _norm_matmul_out(xc_sc, mu_c, inv_c, dinv_ref, cauW_ref, gc_ref, N)


def _tail_body(accs_ref, accc_ref, dinv_ref, ctxb_ref, caub_ref, boh_ref,
               cW1_ref, cb1_ref, cW2_ref, cb2_ref,
               rW1_ref, rb1_ref, rW2_ref, rb2_ref,
               cl_ref, csl_ref):
    bs = ctxb_ref[...]
    bc = caub_ref[...]
    ps = jnp.zeros((G, HID), jnp.float32)
    pc = jnp.zeros((G, HID), jnp.float32)
    for k in range(N // _CH):
        sl = slice(k * _CH, (k + 1) * _CH)
        d = dinv_ref[sl, :]
        xs = jax.nn.relu(0.5 * d * accs_ref[sl, :] + bs)
        xc = jax.nn.relu(0.5 * d * accc_ref[sl, :] + bc)
        boh = boh_ref[:, sl]
        ps = ps + _dot2(boh, xs)
        pc = pc + _dot2(boh, xc)

    def mlp(h, W1, b1, W2, b2):
        h = _bn_small(h)
        h = jax.nn.relu(_dot(h, W1) + b1)
        h = _bn_small(h)
        return _dot(h, W2) + b2

    cl_ref[...] = mlp(pc, cW1_ref[...], cb1_ref[...], cW2_ref[...],
                      cb2_ref[...])
    csl_ref[...] = mlp(ps + pc, rW1_ref[...], rb1_ref[...], rW2_ref[...],
                       rb2_ref[...])


_NT = 16
_JB = 4


def _chunks(total, step=128):
    off = 0
    while off < total:
        yield off, min(step, total - off)
        off += step


def _zero_rows(buf, nrows, width):
    z = jnp.zeros((16,), jnp.float32)

    def zb(i, _):
        for c in range(width // 16):
            buf[i, pl.ds(c * 16, 16)] = z
        return 0

    lax.fori_loop(0, nrows, zb, 0)


def _sc_conv(T, Dpad):
    _KB = 8
    nblk = T // _KB
    zrows = Dpad // _NT

    mesh = plsc.VectorSubcoreMesh(core_axis_name="c", subcore_axis_name="s")

    @functools.partial(
        pl.kernel,
        out_type=jax.ShapeDtypeStruct((2, Dpad, HID), jnp.float32),
        mesh=mesh,
        scratch_types=[
            pltpu.VMEM((_KB, 64), jnp.int32),
            pltpu.VMEM((_KB, 64), jnp.int32),
            pltpu.VMEM((2, 64, HID), jnp.float32),
            pltpu.VMEM_SHARED((Dpad, HID), jnp.float32),
            pltpu.SemaphoreType.DMA,
            pltpu.SemaphoreType.DMA,
        ])
    def body(src_hbm, rowi_hbm, coli_hbm, out_hbm, ibuf, cbuf, gbuf,
             acc, sem0, sem1):
        cid = lax.axis_index("c")
        sid = lax.axis_index("s")
        tid = cid * _NT + sid
        sems = (sem0, sem1)
        _zero_rows(gbuf.at[0], 64, HID)
        for off, cs in _chunks(zrows, 64):
            pltpu.sync_copy(gbuf.at[0, 0:cs],
                            acc.at[pl.ds(sid * zrows + off, cs)])
        plsc.subcore_barrier()

        def blk(b, _):
            pltpu.sync_copy(rowi_hbm.at[tid, pl.ds(b * _KB, _KB)], ibuf)
            pltpu.sync_copy(coli_hbm.at[tid, pl.ds(b * _KB, _KB)], cbuf)
            cps = [pltpu.async_copy(src_hbm.at[ibuf.at[j]], gbuf.at[j % 2],
                                    sems[j % 2]) for j in range(2)]
            for j in range(_KB):
                cps[j].wait()
                pltpu.sync_copy(gbuf.at[j % 2], acc.at[cbuf.at[j]],
                                add=True)
                if j + 2 < _KB:
                    cps.append(
                        pltpu.async_copy(src_hbm.at[ibuf.at[j + 2]],
                                         gbuf.at[j % 2], sems[j % 2]))
            return 0

        lax.fori_loop(0, nblk, blk, 0)
        plsc.subcore_barrier()
        for off, cs in _chunks(zrows, 64):
            pltpu.sync_copy(acc.at[pl.ds(sid * zrows + off, cs)],
                            gbuf.at[0, 0:cs])
            pltpu.sync_copy(gbuf.at[0, 0:cs],
                            out_hbm.at[cid, pl.ds(sid * zrows + off, cs)])

    return body


def _sc_hist(T, Dpad):
    nj = T // (2 * _NT)
    nb = nj // _JB
    zlen = Dpad // _NT

    mesh = plsc.VectorSubcoreMesh(core_axis_name="c", subcore_axis_name="s")

    @functools.partial(
        pl.kernel,
        out_type=jax.ShapeDtypeStruct((2, Dpad), jnp.float32),
        mesh=mesh,
        scratch_types=[
            pltpu.VMEM((_JB, 128), jnp.int32),
            pltpu.VMEM((128,), jnp.float32),
            pltpu.VMEM((zlen,), jnp.float32),
            pltpu.VMEM_SHARED((Dpad,), jnp.float32),
        ])
    def body(idx_hbm, out_hbm, cbuf, ones_v, dbuf, acc):
        cid = lax.axis_index("c")
        sid = lax.axis_index("s")
        one = jnp.full((16,), 1.0, jnp.float32)
        zero = jnp.zeros((16,), jnp.float32)
        for c in range(8):
            ones_v[c * 16:(c + 1) * 16] = one

        def zb(i, _):
            dbuf[pl.ds(i * 16, 16)] = zero
            return 0

        lax.fori_loop(0, zlen // 16, zb, 0)
        pltpu.sync_copy(dbuf, acc.at[pl.ds(sid * zlen, zlen)])
        plsc.subcore_barrier()
        base = (cid * _NT + sid) * nj

        def blk(b, _):
            s = base + b * _JB
            pltpu.sync_copy(idx_hbm.at[pl.ds(s, _JB)], cbuf)
            for j in range(_JB):
                pltpu.sync_copy(ones_v, acc.at[cbuf.at[j]], add=True)
            return 0

        lax.fori_loop(0, nb, blk, 0)
        plsc.subcore_barrier()
        pltpu.sync_copy(acc.at[pl.ds(sid * zlen, zlen)], dbuf)
        pltpu.sync_copy(dbuf, out_hbm.at[cid, pl.ds(sid * zlen, zlen)])

    return body


def _pad_idx(a, total, dummy_base, dummy_n=128):
    pad = total - a.shape[0]
    fill = dummy_base + (jnp.arange(pad, dtype=jnp.int32) % dummy_n)
    return jnp.concatenate([a, fill])


_D_HI = 12544
_T_HI = 176
_D_B = 10112
_T_B = 168
_D_H = 10240
_T_H = 327680 // 128

_conv_hi = _sc_conv(_T_HI, _D_HI)
_conv_base = _sc_conv(_T_B, _D_B)
_hist = _sc_hist(_T_H, _D_H)


def kernel(x, edge_index, batch, node2clique, emb, conv_W, conv_b, att_W1,
           att_b1, att_W2, att_b2, ctx_W, ctx_b, cau_W, cau_b, cmlp_W1,
           cmlp_b1, cmlp_W2, cmlp_b2, rmlp_W1, rmlp_b1, rmlp_W2, rmlp_b2):
    x = x.astype(jnp.int32)
    row = edge_index[0].astype(jnp.int32)
    col = edge_index[1].astype(jnp.int32)
    batch = batch.astype(jnp.int32)
    n2c = node2clique.astype(jnp.int32)

    emb_flat = emb.reshape(9 * 120, HID)
    aW2p = jnp.zeros((HID, HID), jnp.float32).at[:, :2].set(att_W2)
    cW2p = jnp.zeros((HID, HID), jnp.float32).at[:, :2].set(cmlp_W2)
    rW2p = jnp.zeros((HID, HID), jnp.float32).at[:, :2].set(rmlp_W2)
    ab2p = jnp.zeros((1, HID), jnp.float32).at[0, :2].set(att_b2)
    cb2p = jnp.zeros((1, HID), jnp.float32).at[0, :2].set(cmlp_b2)
    rb2p = jnp.zeros((1, HID), jnp.float32).at[0, :2].set(rmlp_b2)

    arN = jnp.arange(N, dtype=jnp.int32)
    arNC = jnp.arange(NC, dtype=jnp.int32)
    hi_row = jnp.concatenate([row, arN, N + n2c, arNC])
    hi_col = jnp.concatenate([col, N + n2c, arN, arNC])
    b_row = jnp.concatenate([row, arN, arN])
    b_col = jnp.concatenate([col, arN, arN])

    hi_rowi = _pad_idx(hi_row, 32 * _T_HI * 64, 0).reshape(32, _T_HI, 64)
    hi_coli = _pad_idx(hi_col, 32 * _T_HI * 64, NC,
                       _D_HI - NC).reshape(32, _T_HI, 64)
    b_rowi = _pad_idx(b_row, 32 * _T_B * 64, 0).reshape(32, _T_B, 64)
    b_coli = _pad_idx(b_col, 32 * _T_B * 64, N,
                      _D_B - N).reshape(32, _T_B, 64)
    h_idx = _pad_idx(col, _T_H * 128, N).reshape(_T_H, 128)

    hpart = _hist(h_idx)
    indeg = hpart[0, :N] + hpart[1, :N]
    dinv_b = (1.0 / jnp.sqrt(0.5 * indeg + 1.0))[:, None]

    boh = (jnp.arange(G, dtype=jnp.int32)[:, None]
           == batch[None, :]).astype(jnp.float32)

    h0, cq, csize = _head(x, emb_flat, n2c.reshape(N // _AB, 1, _AB))
    xa = jnp.concatenate([h0, cq], axis=0)
    dinv_hi = (1.0 / jnp.sqrt(
        jnp.concatenate([indeg + 2.0, csize[:, 0] + 1.0])))[:, None]

    def hi_scatter(g):
        o = _conv_hi(g, hi_rowi, hi_coli)
        return o[0, :NC] + o[1, :NC]

    g = _tc(_pre_body, _f32((NC, HID)), xa, dinv_hi, conv_W[0])
    for i in range(2):
        acc = hi_scatter(g)
        xa, g = _tc(_mid_body, (_f32((NC, HID)), _f32((NC, HID))),
                    xa, acc, dinv_hi, conv_b[i][None, :], conv_W[i + 1])
    acc = hi_scatter(g)
    xn, mm, eps = _tc(_mask_body,
                      (_f32((N, HID)), _f32((C, 2)), _f32((N, 1))),
                      xa, acc, dinv_hi, conv_b[2][None, :],
                      att_W1, att_b1[None, :], aW2p, ab2p, boh)
    m0n = mm[:, 0][n2c]
    m1n = mm[:, 1][n2c]
    sc3 = jnp.stack([m0n, m1n, eps[:, 0]], axis=1)
    gs, gc = _tc(
        _prec2_body, (_f32((N, HID)), _f32((N, HID))),
        xn, sc3, dinv_b, ctx_W, cau_W,
        scratch_shapes=(pltpu.VMEM((N, HID), jnp.float32),
                        pltpu.VMEM((N, HID), jnp.float32)))

    os_ = _conv_base(gs, b_rowi, b_coli)
    oc_ = _conv_base(gc, b_rowi, b_coli)
    accs = os_[0, :N] + os_[1, :N]
    accc = oc_[0, :N] + oc_[1, :N]
    cl, csl = _tc(_tail_body, (_f32((G, HID)), _f32((G, HID))),
                  accs, accc, dinv_b, ctx_b[None, :], cau_b[None, :], boh,
                  cmlp_W1, cmlp_b1[None, :], cW2p, cb2p,
                  rmlp_W1, rmlp_b1[None, :], rW2p, rb2p)
    return (cl[:, :2], csl[:, :2])


# speedup vs baseline: 1.0515x; 1.0515x over previous
import functools

import jax
import jax.numpy as jnp
from jax import lax
from jax.experimental import pallas as pl
from jax.experimental.pallas import tpu as pltpu
from jax.experimental.pallas import tpu_sc as plsc

HID = 128
N = 10000
E = 320000
C = 2500
G = 64
NC = N + C

_CH = 1250

_dot = functools.partial(jnp.dot, preferred_element_type=jnp.float32,
                         precision=jax.lax.Precision.HIGHEST)
_dotd = functools.partial(jnp.dot, preferred_element_type=jnp.float32,
                          precision=jax.lax.Precision.DEFAULT)


def _dot2(oh, x):
    xh = x.astype(jnp.bfloat16).astype(jnp.float32)
    return _dotd(oh, xh) + _dotd(oh, x - xh)


def _f32(s):
    return jax.ShapeDtypeStruct(s, jnp.float32)


def _bn_small(x):
    mu = jnp.mean(x, axis=0, keepdims=True)
    var = jnp.mean(x * x, axis=0, keepdims=True) - mu * mu
    return (x - mu) / jnp.sqrt(var + 1e-5) + 1e-4


def _stats(read_chunk, nrows):
    ssum = jnp.zeros((1, HID), jnp.float32)
    ssq = jnp.zeros((1, HID), jnp.float32)
    for k in range(nrows // _CH):
        c = read_chunk(k)
        ssum = ssum + jnp.sum(c, axis=0, keepdims=True)
        ssq = ssq + jnp.sum(c * c, axis=0, keepdims=True)
    mu = ssum / nrows
    var = ssq / nrows - mu * mu
    return mu, 1.0 / jnp.sqrt(var + 1e-5)


_AB = 1000


def _atom_body(x_ref, embf_ref, h_ref):
    iota120 = jax.lax.broadcasted_iota(jnp.int32, (1, 120), 1)
    h = jnp.zeros((_AB, HID), jnp.float32)
    for i in range(9):
        oh = (x_ref[:, i:i + 1] == iota120).astype(jnp.float32)
        h = h + _dot2(oh, embf_ref[i * 120:(i + 1) * 120, :])
    h_ref[...] = h


def _atom(x, emb_flat):
    return pl.pallas_call(
        _atom_body,
        grid=(N // _AB,),
        in_specs=[pl.BlockSpec((_AB, 9), lambda k: (k, 0)),
                  pl.BlockSpec((9 * 120, HID), lambda k: (0, 0))],
        out_specs=pl.BlockSpec((_AB, HID), lambda k: (k, 0)),
        out_shape=_f32((N, HID)),
    )(x, emb_flat)


def _cpool_body(n2c_ref, h_ref, out_ref, cs_ref):
    k = pl.program_id(0)
    iotaC = jax.lax.broadcasted_iota(jnp.int32, (C, 1), 0)
    oh = (iotaC == n2c_ref[0]).astype(jnp.float32)

    @pl.when(k == 0)
    def _():
        out_ref[...] = jnp.zeros((C, HID), jnp.float32)
        cs_ref[...] = jnp.zeros((C, 1), jnp.float32)

    out_ref[...] += _dot2(oh, h_ref[...])
    cs_ref[...] += jnp.sum(oh, axis=1, keepdims=True)


def _cpool(n2c_blocks, h):
    return pl.pallas_call(
        _cpool_body,
        grid=(N // _AB,),
        in_specs=[pl.BlockSpec((1, 1, _AB), lambda k: (k, 0, 0)),
                  pl.BlockSpec((_AB, HID), lambda k: (k, 0))],
        out_specs=[pl.BlockSpec((C, HID), lambda k: (0, 0)),
                   pl.BlockSpec((C, 1), lambda k: (0, 0))],
        out_shape=(_f32((C, HID)), _f32((C, 1))),
    )(n2c_blocks, h)


def _tc(body, out_shape, *args, scratch_shapes=()):
    return pl.pallas_call(body, out_shape=out_shape,
                          scratch_shapes=list(scratch_shapes))(*args)


def _norm_matmul_out(src_ref, mu, inv, dinv_ref, W_ref, g_ref, nrows):
    for k in range(nrows // _CH):
        sl = slice(k * _CH, (k + 1) * _CH)
        xb = (src_ref[sl, :] - mu) * inv + 1e-4
        h1 = _dot(xb, W_ref[...])
        g_ref[sl, :] = dinv_ref[sl, :] * h1


def _pre_body(xa_ref, dinv_ref, W_ref, g_ref):
    mu, inv = _stats(lambda k: xa_ref[k * _CH:(k + 1) * _CH, :], NC)
    _norm_matmul_out(xa_ref, mu, inv, dinv_ref, W_ref, g_ref, NC)


def _mid_body(xa_ref, acc_ref, dinv_ref, bprev_ref, W_ref, xa2_ref, g_ref):
    b = bprev_ref[...]
    for k in range(NC // _CH):
        sl = slice(k * _CH, (k + 1) * _CH)
        xa2_ref[sl, :] = xa_ref[sl, :] + jax.nn.relu(
            dinv_ref[sl, :] * (acc_ref[0, sl, :] + acc_ref[1, sl, :]) + b)
    mu, inv = _stats(lambda k: xa2_ref[k * _CH:(k + 1) * _CH, :], NC)
    _norm_matmul_out(xa2_ref, mu, inv, dinv_ref, W_ref, g_ref, NC)


def _mask_body(xa_ref, acc_ref, dinv_ref, bprev_ref,
               aW1_ref, ab1_ref, aW2_ref, ab2_ref,
               bohn_ref,
               xn_ref, mm_ref, eps_ref):
    b = bprev_ref[...]
    mo_parts = []
    for k in range(NC // _CH):
        sl = slice(k * _CH, (k + 1) * _CH)
        out = xa_ref[sl, :] + jax.nn.relu(
            dinv_ref[sl, :] * (acc_ref[0, sl, :] + acc_ref[1, sl, :]) + b)
        if k < N // _CH:
            xn_ref[sl, :] = out
        else:
            mo_parts.append(out)
    mo = jnp.concatenate(mo_parts, axis=0)
    hm = _bn_small(mo)
    hm = jax.nn.relu(_dot(hm, aW1_ref[...]) + ab1_ref[...])
    hm = _bn_small(hm)
    att = _dot(hm, aW2_ref[...]) + ab2_ref[...]
    a0 = att[:, 0:1]
    a1 = att[:, 1:2]
    m = jnp.maximum(a0, a1)
    e0 = jnp.exp(a0 - m)
    e1 = jnp.exp(a1 - m)
    mm0 = e0 / (e0 + e1)
    mm_ref[...] = jnp.concatenate([mm0, 1.0 - mm0], axis=1)
    gsum = jnp.zeros((G, HID), jnp.float32)
    cnt = jnp.zeros((G, 1), jnp.float32)
    for k in range(N // _CH):
        sl = slice(k * _CH, (k + 1) * _CH)
        gsum = gsum + _dot2(bohn_ref[:, sl], xn_ref[sl, :])
        cnt = cnt + jnp.sum(bohn_ref[:, sl], axis=1, keepdims=True)
    gmean = gsum / jnp.maximum(cnt, 1.0)
    e = jnp.mean(gmean, axis=-1, keepdims=True)
    for k in range(N // _CH):
        sl = slice(k * _CH, (k + 1) * _CH)
        eps_ref[sl, :] = jax.lax.dot_general(
            bohn_ref[:, sl], e, (((0,), (0,)), ((), ())),
            preferred_element_type=jnp.float32,
            precision=jax.lax.Precision.HIGHEST)


def _prec2_body(xn_ref, sc3_ref, dinv_ref, ctxW_ref, cauW_ref,
                gs_ref, gc_ref, xs_sc, xc_sc):
    for k in range(N // _CH):
        sl = slice(k * _CH, (k + 1) * _CH)
        xn = xn_ref[sl, :]
        m0 = sc3_ref[sl, 0:1]
        m1 = sc3_ref[sl, 1:2]
        eps = sc3_ref[sl, 2:3]
        xs_sc[sl, :] = m0 * xn + (1.0 - m0) * eps
        xc_sc[sl, :] = m1 * xn + (1.0 - m1) * eps
    mu_s, inv_s = _stats(lambda k: xs_sc[k * _CH:(k + 1) * _CH, :], N)
    _norm_matmul_out(xs_sc, mu_s, inv_s, dinv_ref, ctxW_ref, gs_ref, N)
    mu_c, inv_c = _stats(lambda k: xc_sc[k * _CH:(k + 1) * _CH, :], N)
    _norm_matmul_out(xc_sc, mu_c, inv_c, dinv_ref, cauW_ref, gc_ref, N)


def _tail_body(accs_ref, accc_ref, dinv_ref, ctxb_ref, caub_ref, boh_ref,
               cW1_ref, cb1_ref, cW2_ref, cb2_ref,
               rW1_ref, rb1_ref, rW2_ref, rb2_ref,
               cl_ref, csl_ref):
    bs = ctxb_ref[...]
    bc = caub_ref[...]
    ps = jnp.zeros((G, HID), jnp.float32)
    pc = jnp.zeros((G, HID), jnp.float32)
    for k in range(N // _CH):
        sl = slice(k * _CH, (k + 1) * _CH)
        d = dinv_ref[sl, :]
        xs = jax.nn.relu(
            0.5 * d * (accs_ref[0, sl, :] + accs_ref[1, sl, :]) + bs)
        xc = jax.nn.relu(
            0.5 * d * (accc_ref[0, sl, :] + accc_ref[1, sl, :]) + bc)
        boh = boh_ref[:, sl]
        ps = ps + _dot2(boh, xs)
        pc = pc + _dot2(boh, xc)

    def mlp(h, W1, b1, W2, b2):
        h = _bn_small(h)
        h = jax.nn.relu(_dot(h, W1) + b1)
        h = _bn_small(h)
        return _dot(h, W2) + b2

    cl_ref[...] = mlp(pc, cW1_ref[...], cb1_ref[...], cW2_ref[...],
                      cb2_ref[...])
    csl_ref[...] = mlp(ps + pc, rW1_ref[...], rb1_ref[...], rW2_ref[...],
                       rb2_ref[...])


_NT = 16
_JB = 4


def _chunks(total, step=128):
    off = 0
    while off < total:
        yield off, min(step, total - off)
        off += step


def _zero_rows(buf, nrows, width):
    z = jnp.zeros((16,), jnp.float32)

    def zb(i, _):
        for c in range(width // 16):
            buf[i, pl.ds(c * 16, 16)] = z
        return 0

    lax.fori_loop(0, nrows, zb, 0)


def _sc_conv(T, Dpad):
    _KB = 8
    nblk = T // _KB
    zrows = Dpad // _NT

    mesh = plsc.VectorSubcoreMesh(core_axis_name="c", subcore_axis_name="s")

    @functools.partial(
        pl.kernel,
        out_type=jax.ShapeDtypeStruct((2, Dpad, HID), jnp.float32),
        mesh=mesh,
        scratch_types=[
            pltpu.VMEM((_KB, 64), jnp.int32),
            pltpu.VMEM((_KB, 64), jnp.int32),
            pltpu.VMEM((2, 64, HID), jnp.float32),
            pltpu.VMEM_SHARED((Dpad, HID), jnp.float32),
            pltpu.SemaphoreType.DMA,
            pltpu.SemaphoreType.DMA,
        ])
    def body(src_hbm, rowi_hbm, coli_hbm, out_hbm, ibuf, cbuf, gbuf,
             acc, sem0, sem1):
        cid = lax.axis_index("c")
        sid = lax.axis_index("s")
        tid = cid * _NT + sid
        sems = (sem0, sem1)
        _zero_rows(gbuf.at[0], 64, HID)
        for off, cs in _chunks(zrows, 64):
            pltpu.sync_copy(gbuf.at[0, 0:cs],
                            acc.at[pl.ds(sid * zrows + off, cs)])
        plsc.subcore_barrier()

        def blk(b, _):
            pltpu.sync_copy(rowi_hbm.at[tid, pl.ds(b * _KB, _KB)], ibuf)
            pltpu.sync_copy(coli_hbm.at[tid, pl.ds(b * _KB, _KB)], cbuf)
            cps = [pltpu.async_copy(src_hbm.at[ibuf.at[j]], gbuf.at[j % 2],
                                    sems[j % 2]) for j in range(2)]
            for j in range(_KB):
                cps[j].wait()
                pltpu.sync_copy(gbuf.at[j % 2], acc.at[cbuf.at[j]],
                                add=True)
                if j + 2 < _KB:
                    cps.append(
                        pltpu.async_copy(src_hbm.at[ibuf.at[j + 2]],
                                         gbuf.at[j % 2], sems[j % 2]))
            return 0

        lax.fori_loop(0, nblk, blk, 0)
        plsc.subcore_barrier()
        for off, cs in _chunks(zrows, 64):
            pltpu.sync_copy(acc.at[pl.ds(sid * zrows + off, cs)],
                            gbuf.at[0, 0:cs])
            pltpu.sync_copy(gbuf.at[0, 0:cs],
                            out_hbm.at[cid, pl.ds(sid * zrows + off, cs)])

    return body


def _sc_hist(T, Dpad):
    nj = T // (2 * _NT)
    nb = nj // _JB
    zlen = Dpad // _NT

    mesh = plsc.VectorSubcoreMesh(core_axis_name="c", subcore_axis_name="s")

    @functools.partial(
        pl.kernel,
        out_type=jax.ShapeDtypeStruct((2, Dpad), jnp.float32),
        mesh=mesh,
        scratch_types=[
            pltpu.VMEM((_JB, 128), jnp.int32),
            pltpu.VMEM((128,), jnp.float32),
            pltpu.VMEM((zlen,), jnp.float32),
            pltpu.VMEM_SHARED((Dpad,), jnp.float32),
        ])
    def body(idx_hbm, out_hbm, cbuf, ones_v, dbuf, acc):
        cid = lax.axis_index("c")
        sid = lax.axis_index("s")
        one = jnp.full((16,), 1.0, jnp.float32)
        zero = jnp.zeros((16,), jnp.float32)
        for c in range(8):
            ones_v[c * 16:(c + 1) * 16] = one

        def zb(i, _):
            dbuf[pl.ds(i * 16, 16)] = zero
            return 0

        lax.fori_loop(0, zlen // 16, zb, 0)
        pltpu.sync_copy(dbuf, acc.at[pl.ds(sid * zlen, zlen)])
        plsc.subcore_barrier()
        base = (cid * _NT + sid) * nj

        def blk(b, _):
            s = base + b * _JB
            pltpu.sync_copy(idx_hbm.at[pl.ds(s, _JB)], cbuf)
            for j in range(_JB):
                pltpu.sync_copy(ones_v, acc.at[cbuf.at[j]], add=True)
            return 0

        lax.fori_loop(0, nb, blk, 0)
        plsc.subcore_barrier()
        pltpu.sync_copy(acc.at[pl.ds(sid * zlen, zlen)], dbuf)
        pltpu.sync_copy(dbuf, out_hbm.at[cid, pl.ds(sid * zlen, zlen)])

    return body


def _pad_idx(a, total, dummy_base, dummy_n=128):
    pad = total - a.shape[0]
    fill = dummy_base + (jnp.arange(pad, dtype=jnp.int32) % dummy_n)
    return jnp.concatenate([a, fill])


_D_HI = 12544
_T_HI = 176
_D_B = 10112
_T_B = 168
_D_H = 10240
_T_H = 327680 // 128

_conv_hi = _sc_conv(_T_HI, _D_HI)
_conv_base = _sc_conv(_T_B, _D_B)
_hist = _sc_hist(_T_H, _D_H)


def kernel(x, edge_index, batch, node2clique, emb, conv_W, conv_b, att_W1,
           att_b1, att_W2, att_b2, ctx_W, ctx_b, cau_W, cau_b, cmlp_W1,
           cmlp_b1, cmlp_W2, cmlp_b2, rmlp_W1, rmlp_b1, rmlp_W2, rmlp_b2):
    x = x.astype(jnp.int32)
    row = edge_index[0].astype(jnp.int32)
    col = edge_index[1].astype(jnp.int32)
    batch = batch.astype(jnp.int32)
    n2c = node2clique.astype(jnp.int32)

    emb_flat = emb.reshape(9 * 120, HID)
    aW2p = jnp.zeros((HID, HID), jnp.float32).at[:, :2].set(att_W2)
    cW2p = jnp.zeros((HID, HID), jnp.float32).at[:, :2].set(cmlp_W2)
    rW2p = jnp.zeros((HID, HID), jnp.float32).at[:, :2].set(rmlp_W2)
    ab2p = jnp.zeros((1, HID), jnp.float32).at[0, :2].set(att_b2)
    cb2p = jnp.zeros((1, HID), jnp.float32).at[0, :2].set(cmlp_b2)
    rb2p = jnp.zeros((1, HID), jnp.float32).at[0, :2].set(rmlp_b2)

    arN = jnp.arange(N, dtype=jnp.int32)
    arNC = jnp.arange(NC, dtype=jnp.int32)
    hi_row = jnp.concatenate([row, arN, N + n2c, arNC])
    hi_col = jnp.concatenate([col, N + n2c, arN, arNC])
    b_row = jnp.concatenate([row, arN, arN])
    b_col = jnp.concatenate([col, arN, arN])

    hi_rowi = _pad_idx(hi_row, 32 * _T_HI * 64, 0).reshape(32, _T_HI, 64)
    hi_coli = _pad_idx(hi_col, 32 * _T_HI * 64, NC,
                       _D_HI - NC).reshape(32, _T_HI, 64)
    b_rowi = _pad_idx(b_row, 32 * _T_B * 64, 0).reshape(32, _T_B, 64)
    b_coli = _pad_idx(b_col, 32 * _T_B * 64, N,
                      _D_B - N).reshape(32, _T_B, 64)
    h_idx = _pad_idx(col, _T_H * 128, N).reshape(_T_H, 128)

    hpart = _hist(h_idx)
    indeg = hpart[0, :N] + hpart[1, :N]
    dinv_b = (1.0 / jnp.sqrt(0.5 * indeg + 1.0))[:, None]

    boh = (jnp.arange(G, dtype=jnp.int32)[:, None]
           == batch[None, :]).astype(jnp.float32)

    h0 = _atom(x, emb_flat)
    cq, csize = _cpool(n2c.reshape(N // _AB, 1, _AB), h0)
    xa = jnp.concatenate([h0, cq], axis=0)
    dinv_hi = (1.0 / jnp.sqrt(
        jnp.concatenate([indeg + 2.0, csize[:, 0] + 1.0])))[:, None]

    def hi_scatter(g):
        return _conv_hi(g, hi_rowi, hi_coli)

    g = _tc(_pre_body, _f32((NC, HID)), xa, dinv_hi, conv_W[0])
    for i in range(2):
        acc = hi_scatter(g)
        xa, g = _tc(_mid_body, (_f32((NC, HID)), _f32((NC, HID))),
                    xa, acc, dinv_hi, conv_b[i][None, :], conv_W[i + 1])
    acc = hi_scatter(g)
    xn, mm, eps = _tc(_mask_body,
                      (_f32((N, HID)), _f32((C, 2)), _f32((N, 1))),
                      xa, acc, dinv_hi, conv_b[2][None, :],
                      att_W1, att_b1[None, :], aW2p, ab2p, boh)
    m0n = mm[:, 0][n2c]
    m1n = mm[:, 1][n2c]
    sc3 = jnp.stack([m0n, m1n, eps[:, 0]], axis=1)
    gs, gc = _tc(
        _prec2_body, (_f32((N, HID)), _f32((N, HID))),
        xn, sc3, dinv_b, ctx_W, cau_W,
        scratch_shapes=(pltpu.VMEM((N, HID), jnp.float32),
                        pltpu.VMEM((N, HID), jnp.float32)))

    accs = _conv_base(gs, b_rowi, b_coli)
    accc = _conv_base(gc, b_rowi, b_coli)
    cl, csl = _tc(_tail_body, (_f32((G, HID)), _f32((G, HID))),
                  accs, accc, dinv_b, ctx_b[None, :], cau_b[None, :], boh,
                  cmlp_W1, cmlp_b1[None, :], cW2p, cb2p,
                  rmlp_W1, rmlp_b1[None, :], rW2p, rb2p)
    return (cl[:, :2], csl[:, :2])


# speedup vs baseline: 1.1313x; 1.0759x over previous
import functools

import jax
import jax.numpy as jnp
from jax import lax
from jax.experimental import pallas as pl
from jax.experimental.pallas import tpu as pltpu
from jax.experimental.pallas import tpu_sc as plsc

HID = 128
N = 10000
E = 320000
C = 2500
G = 64
NC = N + C

_CH = 1250

_dot = functools.partial(jnp.dot, preferred_element_type=jnp.float32,
                         precision=jax.lax.Precision.HIGHEST)
_dotd = functools.partial(jnp.dot, preferred_element_type=jnp.float32,
                          precision=jax.lax.Precision.DEFAULT)


def _dot2(oh, x):
    xh = x.astype(jnp.bfloat16).astype(jnp.float32)
    return _dotd(oh, xh) + _dotd(oh, x - xh)


def _f32(s):
    return jax.ShapeDtypeStruct(s, jnp.float32)


def _bn_small(x):
    mu = jnp.mean(x, axis=0, keepdims=True)
    var = jnp.mean(x * x, axis=0, keepdims=True) - mu * mu
    return (x - mu) / jnp.sqrt(var + 1e-5) + 1e-4


def _stats(read_chunk, nrows):
    ssum = jnp.zeros((1, HID), jnp.float32)
    ssq = jnp.zeros((1, HID), jnp.float32)
    for k in range(nrows // _CH):
        c = read_chunk(k)
        ssum = ssum + jnp.sum(c, axis=0, keepdims=True)
        ssq = ssq + jnp.sum(c * c, axis=0, keepdims=True)
    mu = ssum / nrows
    var = ssq / nrows - mu * mu
    return mu, 1.0 / jnp.sqrt(var + 1e-5)


_AB = 1000


def _atom_body(x_ref, embf_ref, h_ref):
    iota120 = jax.lax.broadcasted_iota(jnp.int32, (1, 120), 1)
    h = jnp.zeros((_AB, HID), jnp.float32)
    for i in range(9):
        oh = (x_ref[:, i:i + 1] == iota120).astype(jnp.float32)
        h = h + _dot2(oh, embf_ref[i * 120:(i + 1) * 120, :])
    h_ref[...] = h


def _atom(x, emb_flat):
    return pl.pallas_call(
        _atom_body,
        grid=(N // _AB,),
        in_specs=[pl.BlockSpec((_AB, 9), lambda k: (k, 0)),
                  pl.BlockSpec((9 * 120, HID), lambda k: (0, 0))],
        out_specs=pl.BlockSpec((_AB, HID), lambda k: (k, 0)),
        out_shape=_f32((N, HID)),
    )(x, emb_flat)


def _cpool_body(n2c_ref, h_ref, out_ref, cs_ref):
    k = pl.program_id(0)
    iotaC = jax.lax.broadcasted_iota(jnp.int32, (C, 1), 0)
    oh = (iotaC == n2c_ref[0]).astype(jnp.float32)

    @pl.when(k == 0)
    def _():
        out_ref[...] = jnp.zeros((C, HID), jnp.float32)
        cs_ref[...] = jnp.zeros((C, 1), jnp.float32)

    out_ref[...] += _dot2(oh, h_ref[...])
    cs_ref[...] += jnp.sum(oh, axis=1, keepdims=True)


def _cpool(n2c_blocks, h):
    return pl.pallas_call(
        _cpool_body,
        grid=(N // _AB,),
        in_specs=[pl.BlockSpec((1, 1, _AB), lambda k: (k, 0, 0)),
                  pl.BlockSpec((_AB, HID), lambda k: (k, 0))],
        out_specs=[pl.BlockSpec((C, HID), lambda k: (0, 0)),
                   pl.BlockSpec((C, 1), lambda k: (0, 0))],
        out_shape=(_f32((C, HID)), _f32((C, 1))),
    )(n2c_blocks, h)


def _tc(body, out_shape, *args, scratch_shapes=()):
    return pl.pallas_call(body, out_shape=out_shape,
                          scratch_shapes=list(scratch_shapes))(*args)


def _norm_matmul_out(src_ref, mu, inv, dinv_ref, W_ref, g_ref, nrows):
    for k in range(nrows // _CH):
        sl = slice(k * _CH, (k + 1) * _CH)
        xb = (src_ref[sl, :] - mu) * inv + 1e-4
        h1 = _dot(xb, W_ref[...])
        g_ref[sl, :] = dinv_ref[sl, :] * h1


def _pre_body(xa_ref, dinv_ref, W_ref, g_ref):
    mu, inv = _stats(lambda k: xa_ref[k * _CH:(k + 1) * _CH, :], NC)
    _norm_matmul_out(xa_ref, mu, inv, dinv_ref, W_ref, g_ref, NC)


def _mid_body(xa_ref, acc_ref, dinv_ref, bprev_ref, W_ref, xa2_ref, g_ref):
    b = bprev_ref[...]
    for k in range(NC // _CH):
        sl = slice(k * _CH, (k + 1) * _CH)
        xa2_ref[sl, :] = xa_ref[sl, :] + jax.nn.relu(
            dinv_ref[sl, :] * (acc_ref[0, sl, :] + acc_ref[1, sl, :]) + b)
    mu, inv = _stats(lambda k: xa2_ref[k * _CH:(k + 1) * _CH, :], NC)
    _norm_matmul_out(xa2_ref, mu, inv, dinv_ref, W_ref, g_ref, NC)


def _mask_body(xa_ref, acc_ref, dinv_ref, bprev_ref,
               aW1_ref, ab1_ref, aW2_ref, ab2_ref,
               bohn_ref,
               xn_ref, mm_ref, eps_ref):
    b = bprev_ref[...]
    mo_parts = []
    for k in range(NC // _CH):
        sl = slice(k * _CH, (k + 1) * _CH)
        out = xa_ref[sl, :] + jax.nn.relu(
            dinv_ref[sl, :] * (acc_ref[0, sl, :] + acc_ref[1, sl, :]) + b)
        if k < N // _CH:
            xn_ref[sl, :] = out
        else:
            mo_parts.append(out)
    mo = jnp.concatenate(mo_parts, axis=0)
    hm = _bn_small(mo)
    hm = jax.nn.relu(_dot(hm, aW1_ref[...]) + ab1_ref[...])
    hm = _bn_small(hm)
    att = _dot(hm, aW2_ref[...]) + ab2_ref[...]
    a0 = att[:, 0:1]
    a1 = att[:, 1:2]
    m = jnp.maximum(a0, a1)
    e0 = jnp.exp(a0 - m)
    e1 = jnp.exp(a1 - m)
    mm0 = e0 / (e0 + e1)
    mm_ref[...] = jnp.concatenate([mm0, 1.0 - mm0], axis=1)
    gsum = jnp.zeros((G, HID), jnp.float32)
    cnt = jnp.zeros((G, 1), jnp.float32)
    for k in range(N // _CH):
        sl = slice(k * _CH, (k + 1) * _CH)
        gsum = gsum + _dot2(bohn_ref[:, sl], xn_ref[sl, :])
        cnt = cnt + jnp.sum(bohn_ref[:, sl], axis=1, keepdims=True)
    gmean = gsum / jnp.maximum(cnt, 1.0)
    e = jnp.mean(gmean, axis=-1, keepdims=True)
    for k in range(N // _CH):
        sl = slice(k * _CH, (k + 1) * _CH)
        eps_ref[sl, :] = jax.lax.dot_general(
            bohn_ref[:, sl], e, (((0,), (0,)), ((), ())),
            preferred_element_type=jnp.float32,
            precision=jax.lax.Precision.HIGHEST)


def _prec2_body(xn_ref, sc3_ref, dinv_ref, ctxW_ref, cauW_ref,
                gs_ref, gc_ref, xs_sc, xc_sc):
    for k in range(N // _CH):
        sl = slice(k * _CH, (k + 1) * _CH)
        xn = xn_ref[sl, :]
        m0 = sc3_ref[sl, 0:1]
        m1 = sc3_ref[sl, 1:2]
        eps = sc3_ref[sl, 2:3]
        xs_sc[sl, :] = m0 * xn + (1.0 - m0) * eps
        xc_sc[sl, :] = m1 * xn + (1.0 - m1) * eps
    mu_s, inv_s = _stats(lambda k: xs_sc[k * _CH:(k + 1) * _CH, :], N)
    _norm_matmul_out(xs_sc, mu_s, inv_s, dinv_ref, ctxW_ref, gs_ref, N)
    mu_c, inv_c = _stats(lambda k: xc_sc[k * _CH:(k + 1) * _CH, :], N)
    _norm_matmul_out(xc_sc, mu_c, inv_c, dinv_ref, cauW_ref, gc_ref, N)


def _tail_body(accs_ref, accc_ref, dinv_ref, ctxb_ref, caub_ref, boh_ref,
               cW1_ref, cb1_ref, cW2_ref, cb2_ref,
               rW1_ref, rb1_ref, rW2_ref, rb2_ref,
               cl_ref, csl_ref):
    bs = ctxb_ref[...]
    bc = caub_ref[...]
    ps = jnp.zeros((G, HID), jnp.float32)
    pc = jnp.zeros((G, HID), jnp.float32)
    for k in range(N // _CH):
        sl = slice(k * _CH, (k + 1) * _CH)
        d = dinv_ref[sl, :]
        xs = jax.nn.relu(
            0.5 * d * (accs_ref[0, sl, :] + accs_ref[1, sl, :]) + bs)
        xc = jax.nn.relu(
            0.5 * d * (accc_ref[0, sl, :] + accc_ref[1, sl, :]) + bc)
        boh = boh_ref[:, sl]
        ps = ps + _dot2(boh, xs)
        pc = pc + _dot2(boh, xc)

    def mlp(h, W1, b1, W2, b2):
        h = _bn_small(h)
        h = jax.nn.relu(_dot(h, W1) + b1)
        h = _bn_small(h)
        return _dot(h, W2) + b2

    cl_ref[...] = mlp(pc, cW1_ref[...], cb1_ref[...], cW2_ref[...],
                      cb2_ref[...])
    csl_ref[...] = mlp(ps + pc, rW1_ref[...], rb1_ref[...], rW2_ref[...],
                       rb2_ref[...])


_NT = 16
_JB = 4


def _chunks(total, step=128):
    off = 0
    while off < total:
        yield off, min(step, total - off)
        off += step


def _zero_rows(buf, nrows, width):
    z = jnp.zeros((16,), jnp.float32)

    def zb(i, _):
        for c in range(width // 16):
            buf[i, pl.ds(c * 16, 16)] = z
        return 0

    lax.fori_loop(0, nrows, zb, 0)


def _sc_conv(T, Dpad):
    _KB = 16
    nblk = T // _KB
    zrows = Dpad // _NT

    mesh = plsc.VectorSubcoreMesh(core_axis_name="c", subcore_axis_name="s")

    @functools.partial(
        pl.kernel,
        out_type=jax.ShapeDtypeStruct((2, Dpad, HID), jnp.float32),
        mesh=mesh,
        scratch_types=[
            pltpu.VMEM((_KB, 64), jnp.int32),
            pltpu.VMEM((_KB, 64), jnp.int32),
            pltpu.VMEM((2, 64, HID), jnp.float32),
            pltpu.VMEM_SHARED((Dpad, HID), jnp.float32),
            pltpu.SemaphoreType.DMA,
            pltpu.SemaphoreType.DMA,
        ])
    def body(src_hbm, rowi_hbm, coli_hbm, out_hbm, ibuf, cbuf, gbuf,
             acc, sem0, sem1):
        cid = lax.axis_index("c")
        sid = lax.axis_index("s")
        tid = cid * _NT + sid
        sems = (sem0, sem1)
        _zero_rows(gbuf.at[0], 64, HID)
        for off, cs in _chunks(zrows, 64):
            pltpu.sync_copy(gbuf.at[0, 0:cs],
                            acc.at[pl.ds(sid * zrows + off, cs)])
        plsc.subcore_barrier()

        def blk(b, _):
            pltpu.sync_copy(rowi_hbm.at[tid, pl.ds(b * _KB, _KB)], ibuf)
            pltpu.sync_copy(coli_hbm.at[tid, pl.ds(b * _KB, _KB)], cbuf)
            cps = [pltpu.async_copy(src_hbm.at[ibuf.at[j]], gbuf.at[j % 2],
                                    sems[j % 2]) for j in range(2)]
            for j in range(_KB):
                cps[j].wait()
                pltpu.sync_copy(gbuf.at[j % 2], acc.at[cbuf.at[j]],
                                add=True)
                if j + 2 < _KB:
                    cps.append(
                        pltpu.async_copy(src_hbm.at[ibuf.at[j + 2]],
                                         gbuf.at[j % 2], sems[j % 2]))
            return 0

        lax.fori_loop(0, nblk, blk, 0)
        plsc.subcore_barrier()
        for off, cs in _chunks(zrows, 64):
            pltpu.sync_copy(acc.at[pl.ds(sid * zrows + off, cs)],
                            gbuf.at[0, 0:cs])
            pltpu.sync_copy(gbuf.at[0, 0:cs],
                            out_hbm.at[cid, pl.ds(sid * zrows + off, cs)])

    return body


def _sc_hist(T, Dpad):
    nj = T // (2 * _NT)
    nb = nj // _JB
    zlen = Dpad // _NT

    mesh = plsc.VectorSubcoreMesh(core_axis_name="c", subcore_axis_name="s")

    @functools.partial(
        pl.kernel,
        out_type=jax.ShapeDtypeStruct((2, Dpad), jnp.float32),
        mesh=mesh,
        scratch_types=[
            pltpu.VMEM((_JB, 128), jnp.int32),
            pltpu.VMEM((128,), jnp.float32),
            pltpu.VMEM((zlen,), jnp.float32),
            pltpu.VMEM_SHARED((Dpad,), jnp.float32),
        ])
    def body(idx_hbm, out_hbm, cbuf, ones_v, dbuf, acc):
        cid = lax.axis_index("c")
        sid = lax.axis_index("s")
        one = jnp.full((16,), 1.0, jnp.float32)
        zero = jnp.zeros((16,), jnp.float32)
        for c in range(8):
            ones_v[c * 16:(c + 1) * 16] = one

        def zb(i, _):
            dbuf[pl.ds(i * 16, 16)] = zero
            return 0

        lax.fori_loop(0, zlen // 16, zb, 0)
        pltpu.sync_copy(dbuf, acc.at[pl.ds(sid * zlen, zlen)])
        plsc.subcore_barrier()
        base = (cid * _NT + sid) * nj

        def blk(b, _):
            s = base + b * _JB
            pltpu.sync_copy(idx_hbm.at[pl.ds(s, _JB)], cbuf)
            for j in range(_JB):
                pltpu.sync_copy(ones_v, acc.at[cbuf.at[j]], add=True)
            return 0

        lax.fori_loop(0, nb, blk, 0)
        plsc.subcore_barrier()
        pltpu.sync_copy(acc.at[pl.ds(sid * zlen, zlen)], dbuf)
        pltpu.sync_copy(dbuf, out_hbm.at[cid, pl.ds(sid * zlen, zlen)])

    return body


def _pad_idx(a, total, dummy_base, dummy_n=128):
    pad = total - a.shape[0]
    fill = dummy_base + (jnp.arange(pad, dtype=jnp.int32) % dummy_n)
    return jnp.concatenate([a, fill])


_D_HI = 12544
_T_HI = 176
_D_B = 10112
_T_B = 168
_D_H = 10240
_T_H = 327680 // 128

_conv_hi = _sc_conv(_T_HI, _D_HI)
_conv_base = _sc_conv(_T_B, _D_B)
_hist = _sc_hist(_T_H, _D_H)


def kernel(x, edge_index, batch, node2clique, emb, conv_W, conv_b, att_W1,
           att_b1, att_W2, att_b2, ctx_W, ctx_b, cau_W, cau_b, cmlp_W1,
           cmlp_b1, cmlp_W2, cmlp_b2, rmlp_W1, rmlp_b1, rmlp_W2, rmlp_b2):
    x = x.astype(jnp.int32)
    row = edge_index[0].astype(jnp.int32)
    col = edge_index[1].astype(jnp.int32)
    batch = batch.astype(jnp.int32)
    n2c = node2clique.astype(jnp.int32)

    emb_flat = emb.reshape(9 * 120, HID)
    aW2p = jnp.zeros((HID, HID), jnp.float32).at[:, :2].set(att_W2)
    cW2p = jnp.zeros((HID, HID), jnp.float32).at[:, :2].set(cmlp_W2)
    rW2p = jnp.zeros((HID, HID), jnp.float32).at[:, :2].set(rmlp_W2)
    ab2p = jnp.zeros((1, HID), jnp.float32).at[0, :2].set(att_b2)
    cb2p = jnp.zeros((1, HID), jnp.float32).at[0, :2].set(cmlp_b2)
    rb2p = jnp.zeros((1, HID), jnp.float32).at[0, :2].set(rmlp_b2)

    arN = jnp.arange(N, dtype=jnp.int32)
    arNC = jnp.arange(NC, dtype=jnp.int32)
    hi_row = jnp.concatenate([row, arN, N + n2c, arNC])
    hi_col = jnp.concatenate([col, N + n2c, arN, arNC])
    b_row = jnp.concatenate([row, arN, arN])
    b_col = jnp.concatenate([col, arN, arN])

    hi_rowi = _pad_idx(hi_row, 32 * _T_HI * 64, 0).reshape(32, _T_HI, 64)
    hi_coli = _pad_idx(hi_col, 32 * _T_HI * 64, NC,
                       _D_HI - NC).reshape(32, _T_HI, 64)
    b_rowi = _pad_idx(b_row, 32 * _T_B * 64, 0).reshape(32, _T_B, 64)
    b_coli = _pad_idx(b_col, 32 * _T_B * 64, N,
                      _D_B - N).reshape(32, _T_B, 64)
    h_idx = _pad_idx(col, _T_H * 128, N).reshape(_T_H, 128)

    hpart = _hist(h_idx)
    indeg = hpart[0, :N] + hpart[1, :N]
    dinv_b = (1.0 / jnp.sqrt(0.5 * indeg + 1.0))[:, None]

    boh = (jnp.arange(G, dtype=jnp.int32)[:, None]
           == batch[None, :]).astype(jnp.float32)

    h0 = _atom(x, emb_flat)
    cq, csize = _cpool(n2c.reshape(N // _AB, 1, _AB), h0)
    xa = jnp.concatenate([h0, cq], axis=0)
    dinv_hi = (1.0 / jnp.sqrt(
        jnp.concatenate([indeg + 2.0, csize[:, 0] + 1.0])))[:, None]

    def hi_scatter(g):
        return _conv_hi(g, hi_rowi, hi_coli)

    g = _tc(_pre_body, _f32((NC, HID)), xa, dinv_hi, conv_W[0])
    for i in range(2):
        acc = hi_scatter(g)
        xa, g = _tc(_mid_body, (_f32((NC, HID)), _f32((NC, HID))),
                    xa, acc, dinv_hi, conv_b[i][None, :], conv_W[i + 1])
    acc = hi_scatter(g)
    xn, mm, eps = _tc(_mask_body,
                      (_f32((N, HID)), _f32((C, 2)), _f32((N, 1))),
                      xa, acc, dinv_hi, conv_b[2][None, :],
                      att_W1, att_b1[None, :], aW2p, ab2p, boh)
    m0n = mm[:, 0][n2c]
    m1n = mm[:, 1][n2c]
    sc3 = jnp.stack([m0n, m1n, eps[:, 0]], axis=1)
    gs, gc = _tc(
        _prec2_body, (_f32((N, HID)), _f32((N, HID))),
        xn, sc3, dinv_b, ctx_W, cau_W,
        scratch_shapes=(pltpu.VMEM((N, HID), jnp.float32),
                        pltpu.VMEM((N, HID), jnp.float32)))

    accs = _conv_base(gs, b_rowi, b_coli)
    accc = _conv_base(gc, b_rowi, b_coli)
    cl, csl = _tc(_tail_body, (_f32((G, HID)), _f32((G, HID))),
                  accs, accc, dinv_b, ctx_b[None, :], cau_b[None, :], boh,
                  cmlp_W1, cmlp_b1[None, :], cW2p, cb2p,
                  rmlp_W1, rmlp_b1[None, :], rW2p, rb2p)
    return (cl[:, :2], csl[:, :2])
